# baseline (speedup 1.0000x reference)
import jax
import jax.numpy as jnp
from jax import lax
from jax.experimental import pallas as pl
from jax.experimental.pallas import tpu as pltpu
from jax.experimental.pallas import tpu_sc as plsc

BATCH = 16384
DIM = 64
NUM_CORES = 2
NUM_SUBCORES = 16
NUM_WORKERS = NUM_CORES * NUM_SUBCORES
B_PER_W = BATCH // NUM_WORKERS
IDX_CHUNK = 128
N_CHUNKS = B_PER_W // IDX_CHUNK
LANES = 16
D_CHUNKS = DIM // LANES
ROW_GROUPS = B_PER_W // LANES


def _bpr_body(user_idx_hbm, item_idx_hbm, user_table_hbm, item_table_hbm,
              out_hbm, idx_u, idx_i, u_rows, i_rows, out_v, sem):
    wid = lax.axis_index("s") * NUM_CORES + lax.axis_index("c")
    base = wid * B_PER_W

    for j in range(N_CHUNKS):
        off = base + j * IDX_CHUNK
        pltpu.sync_copy(user_idx_hbm.at[pl.ds(off, IDX_CHUNK)], idx_u.at[j])
        pltpu.sync_copy(item_idx_hbm.at[pl.ds(off, IDX_CHUNK)], idx_i.at[j])

    copies = []
    for j in range(N_CHUNKS):
        dst = u_rows.at[pl.ds(j * IDX_CHUNK, IDX_CHUNK)]
        copies.append(pltpu.async_copy(user_table_hbm.at[idx_u.at[j]], dst, sem))
        dst = i_rows.at[pl.ds(j * IDX_CHUNK, IDX_CHUNK)]
        copies.append(pltpu.async_copy(item_table_hbm.at[idx_i.at[j]], dst, sem))
    for c in copies:
        c.wait()

    lane = lax.broadcasted_iota(jnp.int32, (LANES,), 0)
    perms = [lane ^ sh for sh in (8, 4, 2, 1)]

    def group(g, carry):
        acc = jnp.zeros((LANES,), jnp.float32)
        for j in range(LANES):
            r = g * LANES + j
            s = u_rows[r, pl.ds(0, LANES)] * i_rows[r, pl.ds(0, LANES)]
            for c in range(1, D_CHUNKS):
                s = s + u_rows[r, pl.ds(c * LANES, LANES)] * i_rows[r, pl.ds(c * LANES, LANES)]
            for p in perms:
                s = s + s.at[p].get(mode="promise_in_bounds")
            acc = jnp.where(lane == j, s, acc)
        out_v[pl.ds(g * LANES, LANES)] = acc
        return carry

    lax.fori_loop(0, ROW_GROUPS, group, 0)

    pltpu.sync_copy(out_v, out_hbm.at[pl.ds(base, B_PER_W)])


@jax.jit
def kernel(user_idx, item_idx, user_table, item_table):
    mesh = plsc.VectorSubcoreMesh(core_axis_name="c", subcore_axis_name="s",
                                  num_cores=NUM_CORES, num_subcores=NUM_SUBCORES)
    run = pl.kernel(
        _bpr_body,
        out_type=jax.ShapeDtypeStruct((BATCH,), jnp.float32),
        mesh=mesh,
        compiler_params=pltpu.CompilerParams(use_tc_tiling_on_sc=False),
        scratch_types=[
            pltpu.VMEM((N_CHUNKS, IDX_CHUNK), jnp.int32),
            pltpu.VMEM((N_CHUNKS, IDX_CHUNK), jnp.int32),
            pltpu.VMEM((B_PER_W, DIM), jnp.float32),
            pltpu.VMEM((B_PER_W, DIM), jnp.float32),
            pltpu.VMEM((B_PER_W,), jnp.float32),
            pltpu.SemaphoreType.DMA,
        ],
    )
    return run(user_idx, item_idx, user_table, item_table)


# speedup vs baseline: 1.5466x; 1.5466x over previous
import jax
import jax.numpy as jnp
from jax import lax
from jax.experimental import pallas as pl
from jax.experimental.pallas import tpu as pltpu
from jax.experimental.pallas import tpu_sc as plsc

BATCH = 16384
DIM = 64
NUM_CORES = 2
NUM_SUBCORES = 16
NUM_WORKERS = NUM_CORES * NUM_SUBCORES
B_PER_W = BATCH // NUM_WORKERS
IDX_CHUNK = 128
N_IDX_CHUNKS = B_PER_W // IDX_CHUNK
LANES = 16
D_CHUNKS = DIM // LANES
WAVE = 64
N_WAVES = B_PER_W // WAVE


def _bpr_body(user_idx_hbm, item_idx_hbm, user_table, item_table, out_hbm,
              idx_u, idx_i, u_rows, i_rows, out_v, sem_u, sem_i):
    wid = lax.axis_index("s") * NUM_CORES + lax.axis_index("c")
    base = wid * B_PER_W

    for j in range(N_IDX_CHUNKS):
        off = base + j * IDX_CHUNK
        pltpu.sync_copy(user_idx_hbm.at[pl.ds(off, IDX_CHUNK)], idx_u.at[j])
        pltpu.sync_copy(item_idx_hbm.at[pl.ds(off, IDX_CHUNK)], idx_i.at[j])

    lane = lax.broadcasted_iota(jnp.int32, (LANES,), 0)
    perms = [lane ^ sh for sh in (8, 4, 2, 1)]

    def wave(w, carry):
        copies = []
        for g in range(WAVE // LANES):
            w0 = w * WAVE + g * LANES
            uvec = idx_u[w0 // IDX_CHUNK, pl.ds(w0 % IDX_CHUNK, LANES)]
            ivec = idx_i[w0 // IDX_CHUNK, pl.ds(w0 % IDX_CHUNK, LANES)]
            for j in range(LANES):
                t = g * LANES + j
                dst = u_rows.at[t]
                copies.append(pltpu.async_copy(user_table.at[uvec[j]], dst, sem_u))
                dst = i_rows.at[t]
                copies.append(pltpu.async_copy(item_table.at[ivec[j]], dst, sem_i))
        for c in copies:
            c.wait()

        for g in range(WAVE // LANES):
            w0 = w * WAVE + g * LANES
            acc = jnp.zeros((LANES,), jnp.float32)
            for j in range(LANES):
                t = g * LANES + j
                s = (u_rows[t, pl.ds(0, LANES)] * i_rows[t, pl.ds(0, LANES)])
                for d in range(1, D_CHUNKS):
                    s = s + (u_rows[t, pl.ds(d * LANES, LANES)]
                             * i_rows[t, pl.ds(d * LANES, LANES)])
                for q in perms:
                    s = s + s.at[q].get(mode="promise_in_bounds")
                acc = jnp.where(lane == j, s, acc)
            out_v[pl.ds(w * WAVE + g * LANES, LANES)] = acc
        return carry

    lax.fori_loop(0, N_WAVES, wave, 0)

    pltpu.sync_copy(out_v, out_hbm.at[pl.ds(base, B_PER_W)])


@jax.jit
def kernel(user_idx, item_idx, user_table, item_table):
    mesh = plsc.VectorSubcoreMesh(core_axis_name="c", subcore_axis_name="s",
                                  num_cores=NUM_CORES, num_subcores=NUM_SUBCORES)
    run = pl.kernel(
        _bpr_body,
        out_type=jax.ShapeDtypeStruct((BATCH,), jnp.float32),
        mesh=mesh,
        compiler_params=pltpu.CompilerParams(use_tc_tiling_on_sc=True),
        scratch_types=[
            pltpu.VMEM((N_IDX_CHUNKS, IDX_CHUNK), jnp.int32),
            pltpu.VMEM((N_IDX_CHUNKS, IDX_CHUNK), jnp.int32),
            pltpu.VMEM((WAVE, DIM), jnp.float32),
            pltpu.VMEM((WAVE, DIM), jnp.float32),
            pltpu.VMEM((B_PER_W,), jnp.float32),
            pltpu.SemaphoreType.DMA,
            pltpu.SemaphoreType.DMA,
        ],
    )
    return run(user_idx, item_idx, user_table, item_table)
